# baseline (speedup 1.0000x reference)
import functools

import jax
import jax.numpy as jnp
from jax import lax
from jax.experimental import pallas as pl
from jax.experimental.pallas import tpu as pltpu
from jax.experimental.pallas import tpu_sc as plsc

VOCAB = 1000000
EMBED = 64
HIDDEN = 128
CLASSES = 5
BATCH = 4096
SEQ = 200

CHUNKS = ((0, 104), (104, 96))
CHUNK = 104
NC = 2
NS = 16
NW = NC * NS
BPW = BATCH // NW
NV = EMBED // 16


def _pool_body(idx_hbm, table_hbm, out_hbm, idx_v, rows_v, acc_v, sem0, sem1):
    cid = lax.axis_index("c")
    sid = lax.axis_index("s")
    wid = sid * NC + cid
    obase = wid * BPW

    pltpu.sync_copy(idx_hbm.at[pl.ds(obase, BPW)], idx_v)

    def start(b, h, buf, sem):
        off, n = CHUNKS[h]
        pltpu.async_copy(
            table_hbm.at[idx_v.at[b, pl.ds(off, n)]],
            rows_v.at[buf, pl.ds(0, n)],
            sem,
        )

    def wait(b, h, buf, sem):
        off, n = CHUNKS[h]
        pltpu.make_async_copy(
            table_hbm.at[idx_v.at[b, pl.ds(off, n)]],
            rows_v.at[buf, pl.ds(0, n)],
            sem,
        ).wait()

    def sum_chunk(h, buf, acc):
        n = CHUNKS[h][1]

        def rbody(i, acc):
            accs = list(acc)
            for u in range(8):
                r = i * 8 + u
                lo = plsc.bitcast(rows_v[buf, r, pl.ds(0, 16)], jnp.bfloat16)
                hi = plsc.bitcast(rows_v[buf, r, pl.ds(16, 16)], jnp.bfloat16)
                a0, a1 = plsc.unpack(lo, format=plsc.PackFormat.INTERLEAVED)
                b0, b1 = plsc.unpack(hi, format=plsc.PackFormat.INTERLEAVED)
                accs[0] = accs[0] + a0
                accs[1] = accs[1] + b0
                accs[2] = accs[2] + a1
                accs[3] = accs[3] + b1
            return tuple(accs)

        return lax.fori_loop(0, n // 8, rbody, acc)

    start(0, 0, 0, sem0)
    start(0, 1, 1, sem1)

    def gbody(g, carry):
        zero = jnp.zeros((16,), jnp.float32)
        acc = (zero,) * NV
        wait(g, 0, 0, sem0)
        acc = sum_chunk(0, 0, acc)

        @pl.when(g < BPW - 1)
        def _():
            start(g + 1, 0, 0, sem0)

        wait(g, 1, 1, sem1)
        acc = sum_chunk(1, 1, acc)

        @pl.when(g < BPW - 1)
        def _():
            start(g + 1, 1, 1, sem1)

        for j in range(NV):
            acc_v[g, pl.ds(j * 16, 16)] = acc[j]
        return carry

    lax.fori_loop(0, BPW, gbody, 0)
    pltpu.sync_copy(acc_v, out_hbm.at[pl.ds(obase, BPW)])


@functools.partial(jax.jit, static_argnames=())
def _sc_pool(idx2, table):
    mesh = plsc.VectorSubcoreMesh(core_axis_name="c", subcore_axis_name="s")
    return pl.kernel(
        _pool_body,
        out_type=jax.ShapeDtypeStruct((BATCH, EMBED), jnp.float32),
        mesh=mesh,
        scratch_types=[
            pltpu.VMEM((BPW, SEQ), jnp.int32),
            pltpu.VMEM((2, CHUNK, EMBED // 2), jnp.float32),
            pltpu.VMEM((BPW, EMBED), jnp.float32),
            pltpu.SemaphoreType.DMA,
            pltpu.SemaphoreType.DMA,
        ],
        compiler_params=pltpu.CompilerParams(
            use_tc_tiling_on_sc=False, needs_layout_passes=False
        ),
        name="cbow_pool_sc",
    )(idx2, table)


VB = 4096


def _lin_body(t_ref, o_ref):
    xb = t_ref[...].astype(jnp.bfloat16)
    eye = (
        jax.lax.broadcasted_iota(jnp.int32, (EMBED, EMBED), 0)
        == jax.lax.broadcasted_iota(jnp.int32, (EMBED, EMBED), 1)
    ).astype(jnp.bfloat16)
    y = jax.lax.dot_general(
        xb, eye, (((0,), (0,)), ((), ())), preferred_element_type=jnp.float32
    )
    lo = jax.lax.bitcast_convert_type(y[:, 0:32], jnp.uint32)
    hi = jax.lax.bitcast_convert_type(y[:, 32:64], jnp.uint32)
    w = (lo >> 16) | hi
    wf = jax.lax.bitcast_convert_type(w, jnp.float32)
    q = VB // 4
    o_ref[:, 0:32] = wf[0:q, :]
    o_ref[:, 32:64] = wf[q : 2 * q, :]
    o_ref[:, 64:96] = wf[2 * q : 3 * q, :]
    o_ref[:, 96:128] = wf[3 * q : 4 * q, :]


N_VBLK = (VOCAB + VB - 1) // VB
VOCAB_PAD = N_VBLK * VB


def _tc_linearize(table_t):
    return pl.pallas_call(
        _lin_body,
        grid=(N_VBLK,),
        in_specs=[pl.BlockSpec((EMBED, VB), lambda i: (0, i))],
        out_specs=pl.BlockSpec((VB // 4, 128), lambda i: (i, 0)),
        out_shape=jax.ShapeDtypeStruct((VOCAB_PAD // 4, 128), jnp.float32),
    )(table_t)


def _mlp_body(x_ref, w1_ref, b1_ref, w2_ref, b2_ref, w3_ref, b3_ref, o_ref):
    x = x_ref[...]
    h1 = jnp.tanh(
        jnp.dot(x, w1_ref[...], preferred_element_type=jnp.float32) + b1_ref[...]
    )
    h2 = jnp.tanh(
        jnp.dot(h1, w2_ref[...], preferred_element_type=jnp.float32) + b2_ref[...]
    )
    o_ref[...] = (
        jnp.dot(h2, w3_ref[...], preferred_element_type=jnp.float32) + b3_ref[...]
    )


def _tc_mlp(pooled, W1, b1, W2, b2, W3, b3):
    blk = 1024
    grid = (BATCH // blk,)
    full = lambda shape: pl.BlockSpec(shape, lambda i: (0,) * len(shape))
    return pl.pallas_call(
        _mlp_body,
        grid=grid,
        in_specs=[
            pl.BlockSpec((blk, EMBED), lambda i: (i, 0)),
            full((EMBED, HIDDEN)),
            full((1, HIDDEN)),
            full((HIDDEN, HIDDEN)),
            full((1, HIDDEN)),
            full((HIDDEN, CLASSES)),
            full((1, CLASSES)),
        ],
        out_specs=pl.BlockSpec((blk, CLASSES), lambda i: (i, 0)),
        out_shape=jax.ShapeDtypeStruct((BATCH, CLASSES), jnp.float32),
    )(pooled, W1, b1, W2, b2, W3, b3)


def kernel(inputs, embed_table, W1, b1, W2, b2, W3, b3):
    flat = _tc_linearize(embed_table.T)
    qbits = (VB // 4).bit_length() - 1
    idx_lin = (
        (inputs & ~(VB - 1))
        + ((inputs & (VB // 4 - 1)) << 2)
        + ((inputs >> qbits) & 3)
    )
    pooled = _sc_pool(idx_lin, flat.reshape(VOCAB_PAD, EMBED // 2))
    return _tc_mlp(
        pooled,
        W1,
        b1.reshape(1, HIDDEN),
        W2,
        b2.reshape(1, HIDDEN),
        W3,
        b3.reshape(1, CLASSES),
    )


# speedup vs baseline: 1.1518x; 1.1518x over previous
import functools

import jax
import jax.numpy as jnp
from jax import lax
from jax.experimental import pallas as pl
from jax.experimental.pallas import tpu as pltpu
from jax.experimental.pallas import tpu_sc as plsc

VOCAB = 1000000
EMBED = 64
HIDDEN = 128
CLASSES = 5
BATCH = 4096
SEQ = 200

CHUNKS = ((0, 104), (104, 96))
CHUNK = 104
NC = 2
NS = 16
NW = NC * NS
BPW = BATCH // NW
NV = EMBED // 16


def _pool_body(idx_hbm, table_hbm, out_hbm, idx_v, rows_v, acc_v, sem0, sem1):
    cid = lax.axis_index("c")
    sid = lax.axis_index("s")
    wid = sid * NC + cid
    obase = wid * BPW

    pltpu.sync_copy(idx_hbm.at[pl.ds(obase, BPW)], idx_v)

    def start(b, h, buf, sem):
        off, n = CHUNKS[h]
        pltpu.async_copy(
            table_hbm.at[idx_v.at[b, pl.ds(off, n)]],
            rows_v.at[buf, pl.ds(0, n)],
            sem,
        )

    def wait(b, h, buf, sem):
        off, n = CHUNKS[h]
        pltpu.make_async_copy(
            table_hbm.at[idx_v.at[b, pl.ds(off, n)]],
            rows_v.at[buf, pl.ds(0, n)],
            sem,
        ).wait()

    def sum_chunk(h, buf, acc):
        n = CHUNKS[h][1]

        def rbody(i, acc):
            accs = list(acc)
            for u in range(8):
                r = i * 8 + u
                lo = plsc.bitcast(rows_v[buf, r, pl.ds(0, 16)], jnp.bfloat16)
                hi = plsc.bitcast(rows_v[buf, r, pl.ds(16, 16)], jnp.bfloat16)
                a0, a1 = plsc.unpack(lo, format=plsc.PackFormat.INTERLEAVED)
                b0, b1 = plsc.unpack(hi, format=plsc.PackFormat.INTERLEAVED)
                accs[0] = accs[0] + a0
                accs[1] = accs[1] + b0
                accs[2] = accs[2] + a1
                accs[3] = accs[3] + b1
            return tuple(accs)

        return lax.fori_loop(0, n // 8, rbody, acc)

    start(0, 0, 0, sem0)
    start(0, 1, 1, sem1)

    def gbody(g, carry):
        zero = jnp.zeros((16,), jnp.float32)
        acc = (zero,) * NV
        wait(g, 0, 0, sem0)
        acc = sum_chunk(0, 0, acc)

        @pl.when(g < BPW - 1)
        def _():
            start(g + 1, 0, 0, sem0)

        wait(g, 1, 1, sem1)
        acc = sum_chunk(1, 1, acc)

        @pl.when(g < BPW - 1)
        def _():
            start(g + 1, 1, 1, sem1)

        for j in range(NV):
            acc_v[g, pl.ds(j * 16, 16)] = acc[j]
        return carry

    lax.fori_loop(0, BPW, gbody, 0)
    pltpu.sync_copy(acc_v, out_hbm.at[pl.ds(obase, BPW)])


@functools.partial(jax.jit, static_argnames=())
def _sc_pool(idx2, table):
    mesh = plsc.VectorSubcoreMesh(core_axis_name="c", subcore_axis_name="s")
    return pl.kernel(
        _pool_body,
        out_type=jax.ShapeDtypeStruct((BATCH, EMBED), jnp.float32),
        mesh=mesh,
        scratch_types=[
            pltpu.VMEM((BPW, SEQ), jnp.int32),
            pltpu.VMEM((2, CHUNK, EMBED // 2), jnp.float32),
            pltpu.VMEM((BPW, EMBED), jnp.float32),
            pltpu.SemaphoreType.DMA,
            pltpu.SemaphoreType.DMA,
        ],
        compiler_params=pltpu.CompilerParams(
            use_tc_tiling_on_sc=False, needs_layout_passes=False
        ),
        name="cbow_pool_sc",
    )(idx2, table)


VB = 4096


def _lin_body(t_ref, o_ref):
    xb = t_ref[...].astype(jnp.bfloat16)
    q4 = VB // 4
    acc = None
    for q in range(4):
        eyeq = (
            jax.lax.broadcasted_iota(jnp.int32, (32, 128), 0) + 32 * q
            == jax.lax.broadcasted_iota(jnp.int32, (32, 128), 1)
        ).astype(jnp.bfloat16)
        xa = xb[0:32, q * q4 : (q + 1) * q4]
        xc = xb[32:64, q * q4 : (q + 1) * q4]
        ya = jax.lax.dot_general(
            xa, eyeq, (((0,), (0,)), ((), ())),
            preferred_element_type=jnp.float32,
        )
        yc = jax.lax.dot_general(
            xc, eyeq, (((0,), (0,)), ((), ())),
            preferred_element_type=jnp.float32,
        )
        wq = (jax.lax.bitcast_convert_type(ya, jnp.uint32) >> 16) | (
            jax.lax.bitcast_convert_type(yc, jnp.uint32)
        )
        acc = wq if acc is None else acc | wq
    o_ref[...] = jax.lax.bitcast_convert_type(acc, jnp.float32)


N_VBLK = (VOCAB + VB - 1) // VB
VOCAB_PAD = N_VBLK * VB


def _tc_linearize(table_t):
    return pl.pallas_call(
        _lin_body,
        grid=(N_VBLK,),
        in_specs=[pl.BlockSpec((EMBED, VB), lambda i: (0, i))],
        out_specs=pl.BlockSpec((VB // 4, 128), lambda i: (i, 0)),
        out_shape=jax.ShapeDtypeStruct((VOCAB_PAD // 4, 128), jnp.float32),
    )(table_t)


def _mlp_body(x_ref, w1_ref, b1_ref, w2_ref, b2_ref, w3_ref, b3_ref, o_ref):
    x = x_ref[...]
    h1 = jnp.tanh(
        jnp.dot(x, w1_ref[...], preferred_element_type=jnp.float32) + b1_ref[...]
    )
    h2 = jnp.tanh(
        jnp.dot(h1, w2_ref[...], preferred_element_type=jnp.float32) + b2_ref[...]
    )
    o_ref[...] = (
        jnp.dot(h2, w3_ref[...], preferred_element_type=jnp.float32) + b3_ref[...]
    )


def _tc_mlp(pooled, W1, b1, W2, b2, W3, b3):
    blk = 1024
    grid = (BATCH // blk,)
    full = lambda shape: pl.BlockSpec(shape, lambda i: (0,) * len(shape))
    return pl.pallas_call(
        _mlp_body,
        grid=grid,
        in_specs=[
            pl.BlockSpec((blk, EMBED), lambda i: (i, 0)),
            full((EMBED, HIDDEN)),
            full((1, HIDDEN)),
            full((HIDDEN, HIDDEN)),
            full((1, HIDDEN)),
            full((HIDDEN, CLASSES)),
            full((1, CLASSES)),
        ],
        out_specs=pl.BlockSpec((blk, CLASSES), lambda i: (i, 0)),
        out_shape=jax.ShapeDtypeStruct((BATCH, CLASSES), jnp.float32),
    )(pooled, W1, b1, W2, b2, W3, b3)


def kernel(inputs, embed_table, W1, b1, W2, b2, W3, b3):
    flat = _tc_linearize(embed_table.T)
    qbits = (VB // 4).bit_length() - 1
    idx_lin = (
        (inputs & ~(VB - 1))
        + ((inputs & (VB // 4 - 1)) << 2)
        + ((inputs >> qbits) & 3)
    )
    pooled = _sc_pool(idx_lin, flat.reshape(VOCAB_PAD, EMBED // 2))
    return _tc_mlp(
        pooled,
        W1,
        b1.reshape(1, HIDDEN),
        W2,
        b2.reshape(1, HIDDEN),
        W3,
        b3.reshape(1, CLASSES),
    )


# speedup vs baseline: 1.3686x; 1.1882x over previous
import functools

import jax
import jax.numpy as jnp
from jax import lax
from jax.experimental import pallas as pl
from jax.experimental.pallas import tpu as pltpu
from jax.experimental.pallas import tpu_sc as plsc

VOCAB = 1000000
EMBED = 64
HIDDEN = 128
CLASSES = 5
BATCH = 4096
SEQ = 200

CHUNKS = ((0, 104), (104, 96))
CHUNK = 104
NC = 2
NS = 16
NW = NC * NS
BPW = BATCH // NW
NV = EMBED // 16


def _pool_body(idx_hbm, table_hbm, out_hbm, idx_v, rows_v, acc_v, sem0, sem1):
    cid = lax.axis_index("c")
    sid = lax.axis_index("s")
    wid = sid * NC + cid
    obase = wid * BPW

    pltpu.sync_copy(idx_hbm.at[pl.ds(obase, BPW)], idx_v)

    def start(b, h, buf, sem):
        off, n = CHUNKS[h]
        pltpu.async_copy(
            table_hbm.at[idx_v.at[b, pl.ds(off, n)]],
            rows_v.at[buf, pl.ds(0, n)],
            sem,
        )

    def wait(b, h, buf, sem):
        off, n = CHUNKS[h]
        pltpu.make_async_copy(
            table_hbm.at[idx_v.at[b, pl.ds(off, n)]],
            rows_v.at[buf, pl.ds(0, n)],
            sem,
        ).wait()

    def sum_chunk(h, buf, acc):
        n = CHUNKS[h][1]

        def rbody(i, acc):
            accs = list(acc)
            for u in range(8):
                r = i * 8 + u
                lo = plsc.bitcast(rows_v[buf, r, pl.ds(0, 16)], jnp.bfloat16)
                hi = plsc.bitcast(rows_v[buf, r, pl.ds(16, 16)], jnp.bfloat16)
                a0, a1 = plsc.unpack(lo, format=plsc.PackFormat.INTERLEAVED)
                b0, b1 = plsc.unpack(hi, format=plsc.PackFormat.INTERLEAVED)
                accs[0] = accs[0] + a0
                accs[1] = accs[1] + b0
                accs[2] = accs[2] + a1
                accs[3] = accs[3] + b1
            return tuple(accs)

        return lax.fori_loop(0, n // 8, rbody, acc)

    start(0, 0, 0, sem0)
    start(0, 1, 1, sem1)

    def gbody(g, carry):
        zero = jnp.zeros((16,), jnp.float32)
        acc = (zero,) * NV
        wait(g, 0, 0, sem0)
        acc = sum_chunk(0, 0, acc)

        @pl.when(g < BPW - 1)
        def _():
            start(g + 1, 0, 0, sem0)

        wait(g, 1, 1, sem1)
        acc = sum_chunk(1, 1, acc)

        @pl.when(g < BPW - 1)
        def _():
            start(g + 1, 1, 1, sem1)

        for j in range(NV):
            acc_v[g, pl.ds(j * 16, 16)] = acc[j]
        return carry

    lax.fori_loop(0, BPW, gbody, 0)
    pltpu.sync_copy(acc_v, out_hbm.at[pl.ds(obase, BPW)])


@functools.partial(jax.jit, static_argnames=())
def _sc_pool(idx2, table):
    mesh = plsc.VectorSubcoreMesh(core_axis_name="c", subcore_axis_name="s")
    return pl.kernel(
        _pool_body,
        out_type=jax.ShapeDtypeStruct((BATCH, EMBED), jnp.float32),
        mesh=mesh,
        scratch_types=[
            pltpu.VMEM((BPW, SEQ), jnp.int32),
            pltpu.VMEM((2, CHUNK, EMBED // 2), jnp.float32),
            pltpu.VMEM((BPW, EMBED), jnp.float32),
            pltpu.SemaphoreType.DMA,
            pltpu.SemaphoreType.DMA,
        ],
        compiler_params=pltpu.CompilerParams(
            use_tc_tiling_on_sc=False, needs_layout_passes=False
        ),
        name="cbow_pool_sc",
    )(idx2, table)


VB = 8192


def _lin_body(t_ref, o_ref):
    xb = t_ref[...].astype(jnp.bfloat16)
    q4 = VB // 4
    acc = None
    for q in range(4):
        eyeq = (
            jax.lax.broadcasted_iota(jnp.int32, (32, 128), 0) + 32 * q
            == jax.lax.broadcasted_iota(jnp.int32, (32, 128), 1)
        ).astype(jnp.bfloat16)
        xa = xb[0:32, q * q4 : (q + 1) * q4]
        xc = xb[32:64, q * q4 : (q + 1) * q4]
        ya = jax.lax.dot_general(
            xa, eyeq, (((0,), (0,)), ((), ())),
            preferred_element_type=jnp.float32,
        )
        yc = jax.lax.dot_general(
            xc, eyeq, (((0,), (0,)), ((), ())),
            preferred_element_type=jnp.float32,
        )
        wq = (jax.lax.bitcast_convert_type(ya, jnp.uint32) >> 16) | (
            jax.lax.bitcast_convert_type(yc, jnp.uint32)
        )
        acc = wq if acc is None else acc | wq
    o_ref[...] = jax.lax.bitcast_convert_type(acc, jnp.float32)


N_VBLK = (VOCAB + VB - 1) // VB
VOCAB_PAD = N_VBLK * VB


def _tc_linearize(table_t):
    return pl.pallas_call(
        _lin_body,
        grid=(N_VBLK,),
        in_specs=[pl.BlockSpec((EMBED, VB), lambda i: (0, i))],
        out_specs=pl.BlockSpec((VB // 4, 128), lambda i: (i, 0)),
        out_shape=jax.ShapeDtypeStruct((VOCAB_PAD // 4, 128), jnp.float32),
    )(table_t)


def _mlp_body(x_ref, w1_ref, b1_ref, w2_ref, b2_ref, w3_ref, b3_ref, o_ref):
    x = x_ref[...]
    h1 = jnp.tanh(
        jnp.dot(x, w1_ref[...], preferred_element_type=jnp.float32) + b1_ref[...]
    )
    h2 = jnp.tanh(
        jnp.dot(h1, w2_ref[...], preferred_element_type=jnp.float32) + b2_ref[...]
    )
    o_ref[...] = (
        jnp.dot(h2, w3_ref[...], preferred_element_type=jnp.float32) + b3_ref[...]
    )


def _tc_mlp(pooled, W1, b1, W2, b2, W3, b3):
    blk = 1024
    grid = (BATCH // blk,)
    full = lambda shape: pl.BlockSpec(shape, lambda i: (0,) * len(shape))
    return pl.pallas_call(
        _mlp_body,
        grid=grid,
        in_specs=[
            pl.BlockSpec((blk, EMBED), lambda i: (i, 0)),
            full((EMBED, HIDDEN)),
            full((1, HIDDEN)),
            full((HIDDEN, HIDDEN)),
            full((1, HIDDEN)),
            full((HIDDEN, CLASSES)),
            full((1, CLASSES)),
        ],
        out_specs=pl.BlockSpec((blk, CLASSES), lambda i: (i, 0)),
        out_shape=jax.ShapeDtypeStruct((BATCH, CLASSES), jnp.float32),
    )(pooled, W1, b1, W2, b2, W3, b3)


def kernel(inputs, embed_table, W1, b1, W2, b2, W3, b3):
    flat = _tc_linearize(embed_table.T)
    qbits = (VB // 4).bit_length() - 1
    idx_lin = (
        (inputs & ~(VB - 1))
        + ((inputs & (VB // 4 - 1)) << 2)
        + ((inputs >> qbits) & 3)
    )
    pooled = _sc_pool(idx_lin, flat.reshape(VOCAB_PAD, EMBED // 2))
    return _tc_mlp(
        pooled,
        W1,
        b1.reshape(1, HIDDEN),
        W2,
        b2.reshape(1, HIDDEN),
        W3,
        b3.reshape(1, CLASSES),
    )


# speedup vs baseline: 1.5262x; 1.1152x over previous
import functools

import jax
import jax.numpy as jnp
from jax import lax
from jax.experimental import pallas as pl
from jax.experimental.pallas import tpu as pltpu
from jax.experimental.pallas import tpu_sc as plsc

VOCAB = 1000000
EMBED = 64
HIDDEN = 128
CLASSES = 5
BATCH = 4096
SEQ = 200

CHUNKS = ((0, 104), (104, 96))
CHUNK = 104
NC = 2
NS = 16
NW = NC * NS
BPW = BATCH // NW
NV = EMBED // 16


def _pool_body(idx_hbm, table_hbm, out_hbm, idx_v, rows_v, acc_v, sem0, sem1):
    cid = lax.axis_index("c")
    sid = lax.axis_index("s")
    wid = sid * NC + cid
    obase = wid * BPW

    pltpu.sync_copy(idx_hbm.at[pl.ds(obase, BPW)], idx_v)

    def start(b, h, buf, sem):
        off, n = CHUNKS[h]
        pltpu.async_copy(
            table_hbm.at[idx_v.at[b, pl.ds(off, n)]],
            rows_v.at[buf, pl.ds(0, n)],
            sem,
        )

    def wait(b, h, buf, sem):
        off, n = CHUNKS[h]
        pltpu.make_async_copy(
            table_hbm.at[idx_v.at[b, pl.ds(off, n)]],
            rows_v.at[buf, pl.ds(0, n)],
            sem,
        ).wait()

    def sum_chunk(h, buf, acc):
        n = CHUNKS[h][1]

        def rbody(i, acc):
            accs = list(acc)
            for u in range(8):
                r = i * 8 + u
                lo = plsc.bitcast(rows_v[buf, r, pl.ds(0, 16)], jnp.bfloat16)
                hi = plsc.bitcast(rows_v[buf, r, pl.ds(16, 16)], jnp.bfloat16)
                a0, a1 = plsc.unpack(lo, format=plsc.PackFormat.INTERLEAVED)
                b0, b1 = plsc.unpack(hi, format=plsc.PackFormat.INTERLEAVED)
                accs[0] = accs[0] + a0
                accs[1] = accs[1] + b0
                accs[2] = accs[2] + a1
                accs[3] = accs[3] + b1
            return tuple(accs)

        return lax.fori_loop(0, n // 8, rbody, acc)

    start(0, 0, 0, sem0)
    start(0, 1, 1, sem1)

    def gbody(g, carry):
        zero = jnp.zeros((16,), jnp.float32)
        acc = (zero,) * NV
        wait(g, 0, 0, sem0)
        acc = sum_chunk(0, 0, acc)

        @pl.when(g < BPW - 1)
        def _():
            start(g + 1, 0, 0, sem0)

        wait(g, 1, 1, sem1)
        acc = sum_chunk(1, 1, acc)

        @pl.when(g < BPW - 1)
        def _():
            start(g + 1, 1, 1, sem1)

        for j in range(NV):
            acc_v[g, pl.ds(j * 16, 16)] = acc[j]
        return carry

    lax.fori_loop(0, BPW, gbody, 0)
    pltpu.sync_copy(acc_v, out_hbm.at[pl.ds(obase, BPW)])


@functools.partial(jax.jit, static_argnames=())
def _sc_pool(idx2, table):
    mesh = plsc.VectorSubcoreMesh(core_axis_name="c", subcore_axis_name="s")
    return pl.kernel(
        _pool_body,
        out_type=jax.ShapeDtypeStruct((BATCH, EMBED), jnp.float32),
        mesh=mesh,
        scratch_types=[
            pltpu.VMEM((BPW, SEQ), jnp.int32),
            pltpu.VMEM((2, CHUNK, EMBED // 2), jnp.float32),
            pltpu.VMEM((BPW, EMBED), jnp.float32),
            pltpu.SemaphoreType.DMA,
            pltpu.SemaphoreType.DMA,
        ],
        compiler_params=pltpu.CompilerParams(
            use_tc_tiling_on_sc=False, needs_layout_passes=False
        ),
        name="cbow_pool_sc",
    )(idx2, table)


VB = 16384


def _lin_body(t_ref, o_ref):
    xb = t_ref[...].astype(jnp.bfloat16)
    q4 = VB // 4
    acc = None
    for q in range(4):
        eyeq = (
            jax.lax.broadcasted_iota(jnp.int32, (32, 128), 0) + 32 * q
            == jax.lax.broadcasted_iota(jnp.int32, (32, 128), 1)
        ).astype(jnp.bfloat16)
        xa = xb[0:32, q * q4 : (q + 1) * q4]
        xc = xb[32:64, q * q4 : (q + 1) * q4]
        ya = jax.lax.dot_general(
            xa, eyeq, (((0,), (0,)), ((), ())),
            preferred_element_type=jnp.float32,
        )
        yc = jax.lax.dot_general(
            xc, eyeq, (((0,), (0,)), ((), ())),
            preferred_element_type=jnp.float32,
        )
        wq = (jax.lax.bitcast_convert_type(ya, jnp.uint32) >> 16) | (
            jax.lax.bitcast_convert_type(yc, jnp.uint32)
        )
        acc = wq if acc is None else acc | wq
    o_ref[...] = jax.lax.bitcast_convert_type(acc, jnp.float32)


N_VBLK = (VOCAB + VB - 1) // VB
VOCAB_PAD = N_VBLK * VB


def _tc_linearize(table_t):
    return pl.pallas_call(
        _lin_body,
        grid=(N_VBLK,),
        in_specs=[pl.BlockSpec((EMBED, VB), lambda i: (0, i))],
        out_specs=pl.BlockSpec((VB // 4, 128), lambda i: (i, 0)),
        out_shape=jax.ShapeDtypeStruct((VOCAB_PAD // 4, 128), jnp.float32),
    )(table_t)


def _mlp_body(x_ref, w1_ref, b1_ref, w2_ref, b2_ref, w3_ref, b3_ref, o_ref):
    x = x_ref[...]
    h1 = jnp.tanh(
        jnp.dot(x, w1_ref[...], preferred_element_type=jnp.float32) + b1_ref[...]
    )
    h2 = jnp.tanh(
        jnp.dot(h1, w2_ref[...], preferred_element_type=jnp.float32) + b2_ref[...]
    )
    o_ref[...] = (
        jnp.dot(h2, w3_ref[...], preferred_element_type=jnp.float32) + b3_ref[...]
    )


def _tc_mlp(pooled, W1, b1, W2, b2, W3, b3):
    blk = 1024
    grid = (BATCH // blk,)
    full = lambda shape: pl.BlockSpec(shape, lambda i: (0,) * len(shape))
    return pl.pallas_call(
        _mlp_body,
        grid=grid,
        in_specs=[
            pl.BlockSpec((blk, EMBED), lambda i: (i, 0)),
            full((EMBED, HIDDEN)),
            full((1, HIDDEN)),
            full((HIDDEN, HIDDEN)),
            full((1, HIDDEN)),
            full((HIDDEN, CLASSES)),
            full((1, CLASSES)),
        ],
        out_specs=pl.BlockSpec((blk, CLASSES), lambda i: (i, 0)),
        out_shape=jax.ShapeDtypeStruct((BATCH, CLASSES), jnp.float32),
    )(pooled, W1, b1, W2, b2, W3, b3)


def kernel(inputs, embed_table, W1, b1, W2, b2, W3, b3):
    flat = _tc_linearize(embed_table.T)
    qbits = (VB // 4).bit_length() - 1
    idx_lin = (
        (inputs & ~(VB - 1))
        + ((inputs & (VB // 4 - 1)) << 2)
        + ((inputs >> qbits) & 3)
    )
    pooled = _sc_pool(idx_lin, flat.reshape(VOCAB_PAD, EMBED // 2))
    return _tc_mlp(
        pooled,
        W1,
        b1.reshape(1, HIDDEN),
        W2,
        b2.reshape(1, HIDDEN),
        W3,
        b3.reshape(1, CLASSES),
    )


# speedup vs baseline: 1.6185x; 1.0605x over previous
import functools

import jax
import jax.numpy as jnp
from jax import lax
from jax.experimental import pallas as pl
from jax.experimental.pallas import tpu as pltpu
from jax.experimental.pallas import tpu_sc as plsc

VOCAB = 1000000
EMBED = 64
HIDDEN = 128
CLASSES = 5
BATCH = 4096
SEQ = 200

CHUNKS = ((0, 104), (104, 96))
CHUNK = 104
NC = 2
NS = 16
NW = NC * NS
BPW = BATCH // NW
NV = EMBED // 16


def _pool_body(idx_hbm, table_hbm, out_hbm, idx_v, rows_v, acc_v, sem0, sem1):
    cid = lax.axis_index("c")
    sid = lax.axis_index("s")
    wid = sid * NC + cid
    obase = wid * BPW

    pltpu.sync_copy(idx_hbm.at[pl.ds(obase, BPW)], idx_v)

    def start(b, h, buf, sem):
        off, n = CHUNKS[h]
        pltpu.async_copy(
            table_hbm.at[idx_v.at[b, pl.ds(off, n)]],
            rows_v.at[buf, pl.ds(0, n)],
            sem,
        )

    def wait(b, h, buf, sem):
        off, n = CHUNKS[h]
        pltpu.make_async_copy(
            table_hbm.at[idx_v.at[b, pl.ds(off, n)]],
            rows_v.at[buf, pl.ds(0, n)],
            sem,
        ).wait()

    def sum_chunk(h, buf, acc):
        n = CHUNKS[h][1]

        def rbody(i, acc):
            accs = list(acc)
            for u in range(8):
                r = i * 8 + u
                lo = plsc.bitcast(rows_v[buf, r, pl.ds(0, 16)], jnp.bfloat16)
                hi = plsc.bitcast(rows_v[buf, r, pl.ds(16, 16)], jnp.bfloat16)
                a0, a1 = plsc.unpack(lo, format=plsc.PackFormat.INTERLEAVED)
                b0, b1 = plsc.unpack(hi, format=plsc.PackFormat.INTERLEAVED)
                accs[0] = accs[0] + a0
                accs[1] = accs[1] + b0
                accs[2] = accs[2] + a1
                accs[3] = accs[3] + b1
            return tuple(accs)

        return lax.fori_loop(0, n // 8, rbody, acc)

    start(0, 0, 0, sem0)
    start(0, 1, 1, sem1)

    def gbody(g, carry):
        zero = jnp.zeros((16,), jnp.float32)
        acc = (zero,) * NV
        wait(g, 0, 0, sem0)
        acc = sum_chunk(0, 0, acc)

        @pl.when(g < BPW - 1)
        def _():
            start(g + 1, 0, 0, sem0)

        wait(g, 1, 1, sem1)
        acc = sum_chunk(1, 1, acc)

        @pl.when(g < BPW - 1)
        def _():
            start(g + 1, 1, 1, sem1)

        for j in range(NV):
            acc_v[g, pl.ds(j * 16, 16)] = acc[j]
        return carry

    lax.fori_loop(0, BPW, gbody, 0)
    pltpu.sync_copy(acc_v, out_hbm.at[pl.ds(obase, BPW)])


@functools.partial(jax.jit, static_argnames=())
def _sc_pool(idx2, table):
    mesh = plsc.VectorSubcoreMesh(core_axis_name="c", subcore_axis_name="s")
    return pl.kernel(
        _pool_body,
        out_type=jax.ShapeDtypeStruct((BATCH, EMBED), jnp.float32),
        mesh=mesh,
        scratch_types=[
            pltpu.VMEM((BPW, SEQ), jnp.int32),
            pltpu.VMEM((2, CHUNK, EMBED // 2), jnp.float32),
            pltpu.VMEM((BPW, EMBED), jnp.float32),
            pltpu.SemaphoreType.DMA,
            pltpu.SemaphoreType.DMA,
        ],
        compiler_params=pltpu.CompilerParams(
            use_tc_tiling_on_sc=False, needs_layout_passes=False
        ),
        name="cbow_pool_sc",
    )(idx2, table)


VB = 32768


def _lin_body(t_ref, o_ref):
    xb = t_ref[...].astype(jnp.bfloat16)
    q4 = VB // 4
    acc = None
    for q in range(4):
        eyeq = (
            jax.lax.broadcasted_iota(jnp.int32, (32, 128), 0) + 32 * q
            == jax.lax.broadcasted_iota(jnp.int32, (32, 128), 1)
        ).astype(jnp.bfloat16)
        xa = xb[0:32, q * q4 : (q + 1) * q4]
        xc = xb[32:64, q * q4 : (q + 1) * q4]
        ya = jax.lax.dot_general(
            xa, eyeq, (((0,), (0,)), ((), ())),
            preferred_element_type=jnp.float32,
        )
        yc = jax.lax.dot_general(
            xc, eyeq, (((0,), (0,)), ((), ())),
            preferred_element_type=jnp.float32,
        )
        wq = (jax.lax.bitcast_convert_type(ya, jnp.uint32) >> 16) | (
            jax.lax.bitcast_convert_type(yc, jnp.uint32)
        )
        acc = wq if acc is None else acc | wq
    o_ref[...] = jax.lax.bitcast_convert_type(acc, jnp.float32)


N_VBLK = (VOCAB + VB - 1) // VB
VOCAB_PAD = N_VBLK * VB


def _tc_linearize(table_t):
    return pl.pallas_call(
        _lin_body,
        grid=(N_VBLK,),
        in_specs=[pl.BlockSpec((EMBED, VB), lambda i: (0, i))],
        out_specs=pl.BlockSpec((VB // 4, 128), lambda i: (i, 0)),
        out_shape=jax.ShapeDtypeStruct((VOCAB_PAD // 4, 128), jnp.float32),
    )(table_t)


def _mlp_body(x_ref, w1_ref, b1_ref, w2_ref, b2_ref, w3_ref, b3_ref, o_ref):
    x = x_ref[...]
    h1 = jnp.tanh(
        jnp.dot(x, w1_ref[...], preferred_element_type=jnp.float32) + b1_ref[...]
    )
    h2 = jnp.tanh(
        jnp.dot(h1, w2_ref[...], preferred_element_type=jnp.float32) + b2_ref[...]
    )
    o_ref[...] = (
        jnp.dot(h2, w3_ref[...], preferred_element_type=jnp.float32) + b3_ref[...]
    )


def _tc_mlp(pooled, W1, b1, W2, b2, W3, b3):
    blk = 1024
    grid = (BATCH // blk,)
    full = lambda shape: pl.BlockSpec(shape, lambda i: (0,) * len(shape))
    return pl.pallas_call(
        _mlp_body,
        grid=grid,
        in_specs=[
            pl.BlockSpec((blk, EMBED), lambda i: (i, 0)),
            full((EMBED, HIDDEN)),
            full((1, HIDDEN)),
            full((HIDDEN, HIDDEN)),
            full((1, HIDDEN)),
            full((HIDDEN, CLASSES)),
            full((1, CLASSES)),
        ],
        out_specs=pl.BlockSpec((blk, CLASSES), lambda i: (i, 0)),
        out_shape=jax.ShapeDtypeStruct((BATCH, CLASSES), jnp.float32),
    )(pooled, W1, b1, W2, b2, W3, b3)


def kernel(inputs, embed_table, W1, b1, W2, b2, W3, b3):
    flat = _tc_linearize(embed_table.T)
    qbits = (VB // 4).bit_length() - 1
    idx_lin = (
        (inputs & ~(VB - 1))
        + ((inputs & (VB // 4 - 1)) << 2)
        + ((inputs >> qbits) & 3)
    )
    pooled = _sc_pool(idx_lin, flat.reshape(VOCAB_PAD, EMBED // 2))
    return _tc_mlp(
        pooled,
        W1,
        b1.reshape(1, HIDDEN),
        W2,
        b2.reshape(1, HIDDEN),
        W3,
        b3.reshape(1, CLASSES),
    )
